# baseline (speedup 1.0000x reference)
import jax
import jax.numpy as jnp
from jax import lax
from jax.experimental import pallas as pl
from jax.experimental.pallas import tpu as pltpu
from jax.experimental.pallas import tpu_sc as plsc

N_PIX = 320000
N_VIEWS = 80000
N_POINTS = 10000
D = 128
NC = 2
NS = 16
NW = NC * NS

VA = N_VIEWS // NW
KA = 500
TA = 160
NBA = TA // 16

PB = 313
NPOINT_PAD = NW * PB
TB = 128
NBB = TB // 16


def _sread(ref, i):
    return ref[pl.ds(i, 16)][0]


def _lower_bound(seg_hbm, n, probe, v):
    def body(_, c):
        l, r = c
        m = (l + r) >> 1
        ma = pl.multiple_of(jnp.minimum((m >> 4) << 4, n - 16), 16)
        pltpu.sync_copy(seg_hbm.at[pl.ds(ma, 16)], probe.at[pl.ds(0, 16)])
        sm = _sread(probe, m - ma)
        active = l < r
        go_right = active & (sm < v)
        l2 = jnp.where(go_right, m + 1, l)
        r2 = jnp.where(active & jnp.logical_not(sm < v), m, r)
        return l2, r2

    l, _ = lax.fori_loop(0, 19, body, (jnp.int32(0), jnp.int32(n)))
    return l


def _pool_sum_body(x_hbm, seg_hbm, sums_hbm, cnt_hbm,
                   acc, cnta, d0, d1, s0, s1, probe, sem0, sem1):
    w = lax.axis_index("s") * NC + lax.axis_index("c")
    vlo = w * VA
    vhi = vlo + VA
    lo = _lower_bound(seg_hbm, N_PIX, probe, vlo)
    hi = _lower_bound(seg_hbm, N_PIX, probe, vhi)
    lo_t = (lo // TA) * TA
    hi_t = ((hi + TA - 1) // TA) * TA
    nt = (hi_t - lo_t) // TA

    zvec = jnp.zeros((16,), jnp.float32)
    ones = jnp.ones((16,), jnp.float32)

    def zero_acc():
        def zb(j, c):
            for u in range(8):
                acc[pl.ds(j * D + u * 16, 16)] = zvec
            cnta[pl.ds(j * 16, 16)] = zvec
            return c
        lax.fori_loop(0, KA, zb, 0)

    zero_acc()

    def copy_window(bse):
        off = pl.multiple_of(bse * D, 16)
        pltpu.sync_copy(acc.at[pl.ds(0, KA * D)], sums_hbm.at[pl.ds(off, KA * D)])
        off2 = pl.multiple_of(bse * 16, 16)
        pltpu.sync_copy(cnta.at[pl.ds(0, KA * 16)], cnt_hbm.at[pl.ds(off2, KA * 16)])

    def zero_windows(bse, nskip):
        def zb(j, c):
            copy_window(bse + j * KA)
            return c
        lax.fori_loop(0, nskip, zb, 0)

    def do_flush(bse, s):
        copy_window(bse)
        zero_acc()
        nskip = (s - bse - KA) // KA
        zero_windows(bse + KA, nskip)
        return bse + KA * (1 + nskip)

    def pix_body(datar, segr):
        def body(i, base):
            s = _sread(segr, i)
            need = (s >= base + KA) & (s < vhi)
            base = lax.cond(need, lambda bse: do_flush(bse, s),
                            lambda bse: bse, base)
            valid = (s >= vlo) & (s < vhi)
            sl = jnp.where(valid, s - base, KA)
            off = sl * D
            vs = [datar[pl.ds(i * D + g * 16, 16)] for g in range(8)]
            for g in range(8):
                plsc.addupdate(acc.at[pl.ds(off + g * 16, 16)], vs[g])
            plsc.addupdate(cnta.at[pl.ds(sl * 16, 16)], ones)
            return base
        return body

    def slow_tile(datar, segr, base):
        return lax.fori_loop(0, TA, pix_body(datar, segr), base)

    def add_pixel(datar, ib, sl):
        off = sl * D
        vs = [datar[pl.ds(ib + g * 16, 16)] for g in range(8)]
        for g in range(8):
            plsc.addupdate(acc.at[pl.ds(off + g * 16, 16)], vs[g])
        plsc.addupdate(cnta.at[pl.ds(sl * 16, 16)], ones)

    def fast_tile(datar, segr, base):
        s_tl = segr[pl.ds(TA - 16, 16)][15]

        def tile_noflush(b):
            @plsc.parallel_loop(0, TA, step=1, unroll=16)
            def _(i):
                s = _sread(segr, i)
                add_pixel(datar, i * D, s - b)
            return b

        def tile_blocks(b):
            def block(bi, bb):
                vec = segr[pl.ds(bi * 16, 16)]
                s_last = vec[15]

                def slowb(b2):
                    pb = pix_body(datar, segr)
                    def inner(j, b3):
                        return pb(bi * 16 + j, b3)
                    return lax.fori_loop(0, 16, inner, b2)

                def fastb(b2):
                    for j in range(16):
                        add_pixel(datar, (bi * 16 + j) * D, vec[j] - b2)
                    return b2

                return lax.cond(s_last >= bb + KA, slowb, fastb, bb)
            return lax.fori_loop(0, NBA, block, b)

        return lax.cond(s_tl < base + KA, tile_noflush, tile_blocks, base)

    def start_tile(t, datar, segr, sem):
        p0 = pl.multiple_of(lo_t + t * TA, 16)
        pltpu.async_copy(x_hbm.at[pl.ds(p0 * D, TA * D)], datar, sem)
        pltpu.async_copy(seg_hbm.at[pl.ds(p0, TA)], segr.at[pl.ds(0, TA)], sem)

    def wait_tile(t, datar, segr, sem):
        p0 = pl.multiple_of(lo_t + t * TA, 16)
        pltpu.make_async_copy(x_hbm.at[pl.ds(p0 * D, TA * D)], datar, sem).wait()
        pltpu.make_async_copy(seg_hbm.at[pl.ds(p0, TA)], segr.at[pl.ds(0, TA)], sem).wait()

    def process_tile(t, datar, segr, base):
        is_edge = (t == 0) | (t == nt - 1)
        return lax.cond(is_edge,
                        lambda b: slow_tile(datar, segr, b),
                        lambda b: fast_tile(datar, segr, b), base)

    @pl.when(nt > 0)
    def _():
        start_tile(0, d0, s0, sem0)

    def tile_body(t, base):
        def even(b):
            wait_tile(t, d0, s0, sem0)
            @pl.when(t + 1 < nt)
            def _():
                start_tile(t + 1, d1, s1, sem1)
            return process_tile(t, d0, s0, b)

        def odd(b):
            wait_tile(t, d1, s1, sem1)
            @pl.when(t + 1 < nt)
            def _():
                start_tile(t + 1, d0, s0, sem0)
            return process_tile(t, d1, s1, b)

        return lax.cond(t % 2 == 0, even, odd, base)

    base = lax.fori_loop(0, nt, tile_body, vlo)

    copy_window(base)
    zero_acc()
    zero_windows(base + KA, (vhi - base - KA) // KA)


def _pool_max_body(xv_hbm, vseg_hbm, pool_hbm, seen_hbm,
                   acc, seenb, d0, d1, s0, s1, probe, sem0, sem1):
    w = lax.axis_index("s") * NC + lax.axis_index("c")
    plo = w * PB
    phi = plo + PB
    lo = _lower_bound(vseg_hbm, N_VIEWS, probe, plo)
    hi = _lower_bound(vseg_hbm, N_VIEWS, probe, phi)
    lo_t = (lo // TB) * TB
    hi_t = ((hi + TB - 1) // TB) * TB
    nt = (hi_t - lo_t) // TB

    ninf = jnp.full((16,), -jnp.inf, jnp.float32)
    zvec = jnp.zeros((16,), jnp.float32)
    ones = jnp.ones((16,), jnp.float32)

    def init_acc(j, c):
        for u in range(8):
            acc[pl.ds(j * D + u * 16, 16)] = ninf
        seenb[pl.ds(j * 16, 16)] = zvec
        return c

    lax.fori_loop(0, PB, init_acc, 0)

    def upd_pixel(datar, ib, sl):
        off = sl * D
        vs = [datar[pl.ds(ib + g * 16, 16)] for g in range(8)]
        curs = [acc[pl.ds(off + g * 16, 16)] for g in range(8)]
        for g in range(8):
            acc[pl.ds(off + g * 16, 16)] = jnp.maximum(curs[g], vs[g])
        seenb[pl.ds(sl * 16, 16)] = ones

    def slow_tile(datar, segr):
        def body(i, c):
            s = _sread(segr, i)
            valid = (s >= plo) & (s < phi)
            sl = jnp.where(valid, s - plo, PB)
            upd_pixel(datar, i * D, sl)
            return c
        lax.fori_loop(0, TB, body, 0)

    def flush_regs(cur, regs):
        @pl.when(cur >= 0)
        def _():
            off = (cur - plo) * D
            curs = [acc[pl.ds(off + g * 16, 16)] for g in range(8)]
            for g in range(8):
                acc[pl.ds(off + g * 16, 16)] = jnp.maximum(curs[g], regs[g])
            seenb[pl.ds((cur - plo) * 16, 16)] = ones

    def fast_tile(datar, segr):
        init = (jnp.int32(-1),) + tuple(ninf for _ in range(8))

        def body(i, carry):
            cur = carry[0]
            regs = carry[1:]
            s = _sread(segr, i)
            pnew = s != cur
            vs = [datar[pl.ds(i * D + g * 16, 16)] for g in range(8)]

            @pl.when(pnew)
            def _():
                flush_regs(cur, regs)

            regs2 = tuple(jnp.where(pnew, vs[g], jnp.maximum(regs[g], vs[g]))
                          for g in range(8))
            return (s,) + regs2

        carry = lax.fori_loop(0, TB, body, init)
        flush_regs(carry[0], carry[1:])

    def start_tile(t, datar, segr, sem):
        p0 = pl.multiple_of(lo_t + t * TB, 16)
        pltpu.async_copy(xv_hbm.at[pl.ds(p0 * D, TB * D)], datar, sem)
        pltpu.async_copy(vseg_hbm.at[pl.ds(p0, TB)], segr.at[pl.ds(0, TB)], sem)

    def wait_tile(t, datar, segr, sem):
        p0 = pl.multiple_of(lo_t + t * TB, 16)
        pltpu.make_async_copy(xv_hbm.at[pl.ds(p0 * D, TB * D)], datar, sem).wait()
        pltpu.make_async_copy(vseg_hbm.at[pl.ds(p0, TB)], segr.at[pl.ds(0, TB)], sem).wait()

    def process_tile(t, datar, segr):
        is_edge = (t == 0) | (t == nt - 1)
        lax.cond(is_edge,
                 lambda _: slow_tile(datar, segr),
                 lambda _: fast_tile(datar, segr), 0)

    @pl.when(nt > 0)
    def _():
        start_tile(0, d0, s0, sem0)

    def tile_body(t, c):
        def even(cc):
            wait_tile(t, d0, s0, sem0)
            @pl.when(t + 1 < nt)
            def _():
                start_tile(t + 1, d1, s1, sem1)
            process_tile(t, d0, s0)
            return cc

        def odd(cc):
            wait_tile(t, d1, s1, sem1)
            @pl.when(t + 1 < nt)
            def _():
                start_tile(t + 1, d0, s0, sem0)
            process_tile(t, d1, s1)
            return cc

        return lax.cond(t % 2 == 0, even, odd, c)

    lax.fori_loop(0, nt, tile_body, 0)

    def blend(j, c):
        sv = seenb[pl.ds(j * 16, 16)]
        seen = sv > 0.0
        for u in range(8):
            val = acc[pl.ds(j * D + u * 16, 16)]
            acc[pl.ds(j * D + u * 16, 16)] = jnp.where(seen, val, 0.0)
        return c

    lax.fori_loop(0, PB, blend, 0)

    offp = pl.multiple_of(plo * D, 16)
    pltpu.sync_copy(acc.at[pl.ds(0, PB * D)], pool_hbm.at[pl.ds(offp, PB * D)])
    offs = pl.multiple_of(plo * 16, 16)
    pltpu.sync_copy(seenb.at[pl.ds(0, PB * 16)], seen_hbm.at[pl.ds(offs, PB * 16)])


def _build():
    mesh = plsc.VectorSubcoreMesh(core_axis_name="c", subcore_axis_name="s",
                                  num_cores=NC, num_subcores=NS)
    pool_sum = pl.kernel(
        _pool_sum_body,
        out_type=(
            jax.ShapeDtypeStruct((N_VIEWS * D,), jnp.float32),
            jax.ShapeDtypeStruct((N_VIEWS * 16,), jnp.float32),
        ),
        mesh=mesh,
        scratch_types=[
            pltpu.VMEM(((KA + 1) * D,), jnp.float32),
            pltpu.VMEM(((KA + 1) * 16,), jnp.float32),
            pltpu.VMEM((TA * D,), jnp.float32),
            pltpu.VMEM((TA * D,), jnp.float32),
            pltpu.VMEM((TA + 16,), jnp.int32),
            pltpu.VMEM((TA + 16,), jnp.int32),
            pltpu.VMEM((32,), jnp.int32),
            pltpu.SemaphoreType.DMA,
            pltpu.SemaphoreType.DMA,
        ],
    )
    pool_max = pl.kernel(
        _pool_max_body,
        out_type=(
            jax.ShapeDtypeStruct((NPOINT_PAD * D,), jnp.float32),
            jax.ShapeDtypeStruct((NPOINT_PAD * 16,), jnp.float32),
        ),
        mesh=mesh,
        scratch_types=[
            pltpu.VMEM(((PB + 1) * D,), jnp.float32),
            pltpu.VMEM(((PB + 1) * 16,), jnp.float32),
            pltpu.VMEM((TB * D,), jnp.float32),
            pltpu.VMEM((TB * D,), jnp.float32),
            pltpu.VMEM((TB + 16,), jnp.int32),
            pltpu.VMEM((TB + 16,), jnp.int32),
            pltpu.VMEM((32,), jnp.int32),
            pltpu.SemaphoreType.DMA,
            pltpu.SemaphoreType.DMA,
        ],
    )
    return pool_sum, pool_max


BM = 2000


def _mm_body(sums_ref, cnt_ref, w_ref, b_ref, out_ref):
    c = cnt_ref[:, :1]
    mean = sums_ref[:] / jnp.maximum(c, 1.0)
    y = jnp.dot(mean, w_ref[:], preferred_element_type=jnp.float32)
    out_ref[:] = y + b_ref[:] * (c > 0.0).astype(jnp.float32)


_mm = pl.pallas_call(
    _mm_body,
    grid=(N_VIEWS // BM,),
    in_specs=[
        pl.BlockSpec((BM, D), lambda i: (i, 0)),
        pl.BlockSpec((BM, 16), lambda i: (i, 0)),
        pl.BlockSpec((D, D), lambda i: (0, 0)),
        pl.BlockSpec((1, D), lambda i: (0, 0)),
    ],
    out_specs=pl.BlockSpec((BM, D), lambda i: (i, 0)),
    out_shape=jax.ShapeDtypeStruct((N_VIEWS, D), jnp.float32),
)


def kernel(x_3d, x_mod, atomic_seg, view_seg, W, b):
    pool_sum, pool_max = _build()
    seg_a = atomic_seg.astype(jnp.int32)
    seg_v = view_seg.astype(jnp.int32)
    sums_flat, cnt_flat = pool_sum(x_mod.reshape(-1), seg_a)
    xv = _mm(sums_flat.reshape(N_VIEWS, D), cnt_flat.reshape(N_VIEWS, 16),
             W, b.reshape(1, D))
    pool_flat, seen_flat = pool_max(xv.reshape(-1), seg_v)
    x_pool = pool_flat.reshape(NPOINT_PAD, D)[:N_POINTS]
    x_seen = seen_flat.reshape(NPOINT_PAD, 16)[:N_POINTS, 0] > 0.0
    out = jnp.concatenate([x_3d, x_pool], axis=1)
    return out, x_seen


# speedup vs baseline: 1.0766x; 1.0766x over previous
import jax
import jax.numpy as jnp
from jax import lax
from jax.experimental import pallas as pl
from jax.experimental.pallas import tpu as pltpu
from jax.experimental.pallas import tpu_sc as plsc

N_PIX = 320000
N_VIEWS = 80000
N_POINTS = 10000
D = 128
NC = 2
NS = 16
NW = NC * NS

VA = N_VIEWS // NW
KA = 500
TA = 160
NBA = TA // 16

PB = 313
NPOINT_PAD = NW * PB
TB = 128
NBB = TB // 16


def _sread(ref, i):
    return ref[pl.ds(i, 16)][0]


def _lower_bound(seg_hbm, n, probe, v):
    def body(_, c):
        l, r = c
        m = (l + r) >> 1
        ma = pl.multiple_of(jnp.minimum((m >> 4) << 4, n - 16), 16)
        pltpu.sync_copy(seg_hbm.at[pl.ds(ma, 16)], probe.at[pl.ds(0, 16)])
        sm = _sread(probe, m - ma)
        active = l < r
        go_right = active & (sm < v)
        l2 = jnp.where(go_right, m + 1, l)
        r2 = jnp.where(active & jnp.logical_not(sm < v), m, r)
        return l2, r2

    l, _ = lax.fori_loop(0, 19, body, (jnp.int32(0), jnp.int32(n)))
    return l


def _pool_sum_body(x_hbm, seg_hbm, sums_hbm, cnt_hbm,
                   acc, cnta, d0, d1, s0, s1, probe, sem0, sem1):
    w = lax.axis_index("s") * NC + lax.axis_index("c")
    vlo = w * VA
    vhi = vlo + VA
    lo = _lower_bound(seg_hbm, N_PIX, probe, vlo)
    hi = _lower_bound(seg_hbm, N_PIX, probe, vhi)
    lo_t = (lo // TA) * TA
    hi_t = ((hi + TA - 1) // TA) * TA
    nt = (hi_t - lo_t) // TA

    zvec = jnp.zeros((16,), jnp.float32)
    ones = jnp.ones((16,), jnp.float32)

    def zero_acc():
        def zb(j, c):
            for u in range(8):
                acc[pl.ds(j * D + u * 16, 16)] = zvec
            cnta[pl.ds(j * 16, 16)] = zvec
            return c
        lax.fori_loop(0, KA, zb, 0)

    zero_acc()

    def copy_window(bse):
        off = pl.multiple_of(bse * D, 16)
        pltpu.sync_copy(acc.at[pl.ds(0, KA * D)], sums_hbm.at[pl.ds(off, KA * D)])
        off2 = pl.multiple_of(bse * 16, 16)
        pltpu.sync_copy(cnta.at[pl.ds(0, KA * 16)], cnt_hbm.at[pl.ds(off2, KA * 16)])

    def zero_windows(bse, nskip):
        def zb(j, c):
            copy_window(bse + j * KA)
            return c
        lax.fori_loop(0, nskip, zb, 0)

    def do_flush(bse, s):
        copy_window(bse)
        zero_acc()
        nskip = (s - bse - KA) // KA
        zero_windows(bse + KA, nskip)
        return bse + KA * (1 + nskip)

    def pix_body(datar, segr):
        def body(i, base):
            s = _sread(segr, i)
            need = (s >= base + KA) & (s < vhi)
            base = lax.cond(need, lambda bse: do_flush(bse, s),
                            lambda bse: bse, base)
            valid = (s >= vlo) & (s < vhi)
            sl = jnp.where(valid, s - base, KA)
            off = sl * D
            vs = [datar[pl.ds(i * D + g * 16, 16)] for g in range(8)]
            for g in range(8):
                plsc.addupdate(acc.at[pl.ds(off + g * 16, 16)], vs[g])
            plsc.addupdate(cnta.at[pl.ds(sl * 16, 16)], ones)
            return base
        return body

    def slow_tile(datar, segr, base):
        return lax.fori_loop(0, TA, pix_body(datar, segr), base)

    def add_pixel(datar, ib, sl):
        off = sl * D
        vs = [datar[pl.ds(ib + g * 16, 16)] for g in range(8)]
        for g in range(8):
            plsc.addupdate(acc.at[pl.ds(off + g * 16, 16)], vs[g])
        plsc.addupdate(cnta.at[pl.ds(sl * 16, 16)], ones)

    def fast_tile(datar, segr, base):
        s_tl = segr[pl.ds(TA - 16, 16)][15]

        def tile_noflush(b):
            @plsc.parallel_loop(0, TA, step=1, unroll=16)
            def _(i):
                s = _sread(segr, i)
                add_pixel(datar, i * D, s - b)
            return b

        def tile_blocks(b):
            def block(bi, bb):
                vec = segr[pl.ds(bi * 16, 16)]
                s_last = vec[15]

                def slowb(b2):
                    pb = pix_body(datar, segr)
                    def inner(j, b3):
                        return pb(bi * 16 + j, b3)
                    return lax.fori_loop(0, 16, inner, b2)

                def fastb(b2):
                    for j in range(16):
                        add_pixel(datar, (bi * 16 + j) * D, vec[j] - b2)
                    return b2

                return lax.cond(s_last >= bb + KA, slowb, fastb, bb)
            return lax.fori_loop(0, NBA, block, b)

        return lax.cond(s_tl < base + KA, tile_noflush, tile_blocks, base)

    def start_tile(t, datar, segr, sem):
        p0 = pl.multiple_of(lo_t + t * TA, 16)
        pltpu.async_copy(x_hbm.at[pl.ds(p0 * D, TA * D)], datar, sem)
        pltpu.async_copy(seg_hbm.at[pl.ds(p0, TA)], segr.at[pl.ds(0, TA)], sem)

    def wait_tile(t, datar, segr, sem):
        p0 = pl.multiple_of(lo_t + t * TA, 16)
        pltpu.make_async_copy(x_hbm.at[pl.ds(p0 * D, TA * D)], datar, sem).wait()
        pltpu.make_async_copy(seg_hbm.at[pl.ds(p0, TA)], segr.at[pl.ds(0, TA)], sem).wait()

    def process_tile(t, datar, segr, base):
        is_edge = (t == 0) | (t == nt - 1)
        return lax.cond(is_edge,
                        lambda b: slow_tile(datar, segr, b),
                        lambda b: fast_tile(datar, segr, b), base)

    @pl.when(nt > 0)
    def _():
        start_tile(0, d0, s0, sem0)

    def tile_body(t, base):
        def even(b):
            wait_tile(t, d0, s0, sem0)
            @pl.when(t + 1 < nt)
            def _():
                start_tile(t + 1, d1, s1, sem1)
            return process_tile(t, d0, s0, b)

        def odd(b):
            wait_tile(t, d1, s1, sem1)
            @pl.when(t + 1 < nt)
            def _():
                start_tile(t + 1, d0, s0, sem0)
            return process_tile(t, d1, s1, b)

        return lax.cond(t % 2 == 0, even, odd, base)

    base = lax.fori_loop(0, nt, tile_body, vlo)

    copy_window(base)
    zero_acc()
    zero_windows(base + KA, (vhi - base - KA) // KA)


def _pool_max_body(xv_hbm, vseg_hbm, pool_hbm, seen_hbm,
                   acc, seenb, d0, d1, s0, s1, probe, sem0, sem1):
    w = lax.axis_index("s") * NC + lax.axis_index("c")
    plo = w * PB
    phi = plo + PB
    lo = _lower_bound(vseg_hbm, N_VIEWS, probe, plo)
    hi = _lower_bound(vseg_hbm, N_VIEWS, probe, phi)
    lo_t = (lo // TB) * TB
    hi_t = ((hi + TB - 1) // TB) * TB
    nt = (hi_t - lo_t) // TB

    ninf = jnp.full((16,), -jnp.inf, jnp.float32)
    zvec = jnp.zeros((16,), jnp.float32)
    ones = jnp.ones((16,), jnp.float32)

    def init_acc(j, c):
        for u in range(8):
            acc[pl.ds(j * D + u * 16, 16)] = ninf
        seenb[pl.ds(j * 16, 16)] = zvec
        return c

    lax.fori_loop(0, PB, init_acc, 0)

    def upd_pixel(datar, ib, sl):
        off = sl * D
        vs = [datar[pl.ds(ib + g * 16, 16)] for g in range(8)]
        curs = [acc[pl.ds(off + g * 16, 16)] for g in range(8)]
        for g in range(8):
            acc[pl.ds(off + g * 16, 16)] = jnp.maximum(curs[g], vs[g])
        seenb[pl.ds(sl * 16, 16)] = ones

    def slow_tile(datar, segr):
        def body(i, c):
            s = _sread(segr, i)
            valid = (s >= plo) & (s < phi)
            sl = jnp.where(valid, s - plo, PB)
            upd_pixel(datar, i * D, sl)
            return c
        lax.fori_loop(0, TB, body, 0)

    def fast_tile(datar, segr):
        def block(bi, c):
            vec = segr[pl.ds(bi * 16, 16)]
            for j in range(16):
                upd_pixel(datar, (bi * 16 + j) * D, vec[j] - plo)
            return c
        lax.fori_loop(0, NBB, block, 0)

    def start_tile(t, datar, segr, sem):
        p0 = pl.multiple_of(lo_t + t * TB, 16)
        pltpu.async_copy(xv_hbm.at[pl.ds(p0 * D, TB * D)], datar, sem)
        pltpu.async_copy(vseg_hbm.at[pl.ds(p0, TB)], segr.at[pl.ds(0, TB)], sem)

    def wait_tile(t, datar, segr, sem):
        p0 = pl.multiple_of(lo_t + t * TB, 16)
        pltpu.make_async_copy(xv_hbm.at[pl.ds(p0 * D, TB * D)], datar, sem).wait()
        pltpu.make_async_copy(vseg_hbm.at[pl.ds(p0, TB)], segr.at[pl.ds(0, TB)], sem).wait()

    def process_tile(t, datar, segr):
        is_edge = (t == 0) | (t == nt - 1)
        lax.cond(is_edge,
                 lambda _: slow_tile(datar, segr),
                 lambda _: fast_tile(datar, segr), 0)

    @pl.when(nt > 0)
    def _():
        start_tile(0, d0, s0, sem0)

    def tile_body(t, c):
        def even(cc):
            wait_tile(t, d0, s0, sem0)
            @pl.when(t + 1 < nt)
            def _():
                start_tile(t + 1, d1, s1, sem1)
            process_tile(t, d0, s0)
            return cc

        def odd(cc):
            wait_tile(t, d1, s1, sem1)
            @pl.when(t + 1 < nt)
            def _():
                start_tile(t + 1, d0, s0, sem0)
            process_tile(t, d1, s1)
            return cc

        return lax.cond(t % 2 == 0, even, odd, c)

    lax.fori_loop(0, nt, tile_body, 0)

    def blend(j, c):
        sv = seenb[pl.ds(j * 16, 16)]
        seen = sv > 0.0
        for u in range(8):
            val = acc[pl.ds(j * D + u * 16, 16)]
            acc[pl.ds(j * D + u * 16, 16)] = jnp.where(seen, val, 0.0)
        return c

    lax.fori_loop(0, PB, blend, 0)

    offp = pl.multiple_of(plo * D, 16)
    pltpu.sync_copy(acc.at[pl.ds(0, PB * D)], pool_hbm.at[pl.ds(offp, PB * D)])
    offs = pl.multiple_of(plo * 16, 16)
    pltpu.sync_copy(seenb.at[pl.ds(0, PB * 16)], seen_hbm.at[pl.ds(offs, PB * 16)])


def _build():
    mesh = plsc.VectorSubcoreMesh(core_axis_name="c", subcore_axis_name="s",
                                  num_cores=NC, num_subcores=NS)
    pool_sum = pl.kernel(
        _pool_sum_body,
        out_type=(
            jax.ShapeDtypeStruct((N_VIEWS * D,), jnp.float32),
            jax.ShapeDtypeStruct((N_VIEWS * 16,), jnp.float32),
        ),
        mesh=mesh,
        scratch_types=[
            pltpu.VMEM(((KA + 1) * D,), jnp.float32),
            pltpu.VMEM(((KA + 1) * 16,), jnp.float32),
            pltpu.VMEM((TA * D,), jnp.float32),
            pltpu.VMEM((TA * D,), jnp.float32),
            pltpu.VMEM((TA + 16,), jnp.int32),
            pltpu.VMEM((TA + 16,), jnp.int32),
            pltpu.VMEM((32,), jnp.int32),
            pltpu.SemaphoreType.DMA,
            pltpu.SemaphoreType.DMA,
        ],
    )
    pool_max = pl.kernel(
        _pool_max_body,
        out_type=(
            jax.ShapeDtypeStruct((NPOINT_PAD * D,), jnp.float32),
            jax.ShapeDtypeStruct((NPOINT_PAD * 16,), jnp.float32),
        ),
        mesh=mesh,
        scratch_types=[
            pltpu.VMEM(((PB + 1) * D,), jnp.float32),
            pltpu.VMEM(((PB + 1) * 16,), jnp.float32),
            pltpu.VMEM((TB * D,), jnp.float32),
            pltpu.VMEM((TB * D,), jnp.float32),
            pltpu.VMEM((TB + 16,), jnp.int32),
            pltpu.VMEM((TB + 16,), jnp.int32),
            pltpu.VMEM((32,), jnp.int32),
            pltpu.SemaphoreType.DMA,
            pltpu.SemaphoreType.DMA,
        ],
    )
    return pool_sum, pool_max


BM = 2000


def _mm_body(sums_ref, cnt_ref, w_ref, b_ref, out_ref):
    c = cnt_ref[:, :1]
    mean = sums_ref[:] / jnp.maximum(c, 1.0)
    y = jnp.dot(mean, w_ref[:], preferred_element_type=jnp.float32)
    out_ref[:] = y + b_ref[:] * (c > 0.0).astype(jnp.float32)


_mm = pl.pallas_call(
    _mm_body,
    grid=(N_VIEWS // BM,),
    in_specs=[
        pl.BlockSpec((BM, D), lambda i: (i, 0)),
        pl.BlockSpec((BM, 16), lambda i: (i, 0)),
        pl.BlockSpec((D, D), lambda i: (0, 0)),
        pl.BlockSpec((1, D), lambda i: (0, 0)),
    ],
    out_specs=pl.BlockSpec((BM, D), lambda i: (i, 0)),
    out_shape=jax.ShapeDtypeStruct((N_VIEWS, D), jnp.float32),
)


def kernel(x_3d, x_mod, atomic_seg, view_seg, W, b):
    pool_sum, pool_max = _build()
    seg_a = atomic_seg.astype(jnp.int32)
    seg_v = view_seg.astype(jnp.int32)
    sums_flat, cnt_flat = pool_sum(x_mod.reshape(-1), seg_a)
    xv = _mm(sums_flat.reshape(N_VIEWS, D), cnt_flat.reshape(N_VIEWS, 16),
             W, b.reshape(1, D))
    pool_flat, seen_flat = pool_max(xv.reshape(-1), seg_v)
    x_pool = pool_flat.reshape(NPOINT_PAD, D)[:N_POINTS]
    x_seen = seen_flat.reshape(NPOINT_PAD, 16)[:N_POINTS, 0] > 0.0
    out = jnp.concatenate([x_3d, x_pool], axis=1)
    return out, x_seen


# speedup vs baseline: 1.1016x; 1.0232x over previous
import jax
import jax.numpy as jnp
from jax import lax
from jax.experimental import pallas as pl
from jax.experimental.pallas import tpu as pltpu
from jax.experimental.pallas import tpu_sc as plsc

N_PIX = 320000
N_VIEWS = 80000
N_POINTS = 10000
D = 128
NC = 2
NS = 16
NW = NC * NS

VA = N_VIEWS // NW
KA = 500
TA = 160
NBA = TA // 16

PB = 313
NPOINT_PAD = NW * PB
TB = 128
NBB = TB // 16


def _sread(ref, i):
    return ref[pl.ds(i, 16)][0]


def _lower_bound(seg_hbm, n, probe, v):
    def body(_, c):
        l, r = c
        m = (l + r) >> 1
        ma = pl.multiple_of(jnp.minimum((m >> 4) << 4, n - 16), 16)
        pltpu.sync_copy(seg_hbm.at[pl.ds(ma, 16)], probe.at[pl.ds(0, 16)])
        sm = _sread(probe, m - ma)
        active = l < r
        go_right = active & (sm < v)
        l2 = jnp.where(go_right, m + 1, l)
        r2 = jnp.where(active & jnp.logical_not(sm < v), m, r)
        return l2, r2

    l, _ = lax.fori_loop(0, 19, body, (jnp.int32(0), jnp.int32(n)))
    return l


def _pool_sum_body(x_hbm, seg_hbm, sums_hbm, cnt_hbm,
                   acc, cnta, d0, d1, s0, s1, probe, sem0, sem1):
    w = lax.axis_index("s") * NC + lax.axis_index("c")
    vlo = w * VA
    vhi = vlo + VA
    lo = _lower_bound(seg_hbm, N_PIX, probe, vlo)
    hi = _lower_bound(seg_hbm, N_PIX, probe, vhi)
    lo_t = (lo // TA) * TA
    hi_t = ((hi + TA - 1) // TA) * TA
    nt = (hi_t - lo_t) // TA

    zvec = jnp.zeros((16,), jnp.float32)
    ones = jnp.ones((16,), jnp.float32)

    def zero_acc():
        def zb(j, c):
            for u in range(8):
                acc[pl.ds(j * D + u * 16, 16)] = zvec
            cnta[pl.ds(j * 16, 16)] = zvec
            return c
        lax.fori_loop(0, KA, zb, 0)

    zero_acc()

    def copy_window(bse):
        off = pl.multiple_of(bse * D, 16)
        pltpu.sync_copy(acc.at[pl.ds(0, KA * D)], sums_hbm.at[pl.ds(off, KA * D)])
        off2 = pl.multiple_of(bse * 16, 16)
        pltpu.sync_copy(cnta.at[pl.ds(0, KA * 16)], cnt_hbm.at[pl.ds(off2, KA * 16)])

    def zero_windows(bse, nskip):
        def zb(j, c):
            copy_window(bse + j * KA)
            return c
        lax.fori_loop(0, nskip, zb, 0)

    def do_flush(bse, s):
        copy_window(bse)
        zero_acc()
        nskip = (s - bse - KA) // KA
        zero_windows(bse + KA, nskip)
        return bse + KA * (1 + nskip)

    def pix_body(datar, segr):
        def body(i, base):
            s = _sread(segr, i)
            need = (s >= base + KA) & (s < vhi)
            base = lax.cond(need, lambda bse: do_flush(bse, s),
                            lambda bse: bse, base)
            valid = (s >= vlo) & (s < vhi)
            sl = jnp.where(valid, s - base, KA)
            off = sl * D
            vs = [datar[pl.ds(i * D + g * 16, 16)] for g in range(8)]
            for g in range(8):
                plsc.addupdate(acc.at[pl.ds(off + g * 16, 16)], vs[g])
            plsc.addupdate(cnta.at[pl.ds(sl * 16, 16)], ones)
            return base
        return body

    def slow_tile(datar, segr, base):
        return lax.fori_loop(0, TA, pix_body(datar, segr), base)

    def add_pixel(datar, ib, sl):
        off = sl * D
        vs = [datar[pl.ds(ib + g * 16, 16)] for g in range(8)]
        for g in range(8):
            plsc.addupdate(acc.at[pl.ds(off + g * 16, 16)], vs[g])
        plsc.addupdate(cnta.at[pl.ds(sl * 16, 16)], ones)

    def fast_tile(datar, segr, base):
        s_tl = segr[pl.ds(TA - 16, 16)][15]

        def tile_noflush(b):
            @plsc.parallel_loop(0, TA, step=1, unroll=16)
            def _(i):
                s = _sread(segr, i)
                add_pixel(datar, i * D, s - b)
            return b

        def tile_blocks(b):
            def block(bi, bb):
                vec = segr[pl.ds(bi * 16, 16)]
                s_last = vec[15]

                def slowb(b2):
                    pb = pix_body(datar, segr)
                    def inner(j, b3):
                        return pb(bi * 16 + j, b3)
                    return lax.fori_loop(0, 16, inner, b2)

                def fastb(b2):
                    for j in range(16):
                        add_pixel(datar, (bi * 16 + j) * D, vec[j] - b2)
                    return b2

                return lax.cond(s_last >= bb + KA, slowb, fastb, bb)
            return lax.fori_loop(0, NBA, block, b)

        return lax.cond(s_tl < base + KA, tile_noflush, tile_blocks, base)

    def start_tile(t, datar, segr, sem):
        p0 = pl.multiple_of(lo_t + t * TA, 16)
        pltpu.async_copy(x_hbm.at[pl.ds(p0 * D, TA * D)], datar, sem)
        pltpu.async_copy(seg_hbm.at[pl.ds(p0, TA)], segr.at[pl.ds(0, TA)], sem)

    def wait_tile(t, datar, segr, sem):
        p0 = pl.multiple_of(lo_t + t * TA, 16)
        pltpu.make_async_copy(x_hbm.at[pl.ds(p0 * D, TA * D)], datar, sem).wait()
        pltpu.make_async_copy(seg_hbm.at[pl.ds(p0, TA)], segr.at[pl.ds(0, TA)], sem).wait()

    def process_tile(t, datar, segr, base):
        is_edge = (t == 0) | (t == nt - 1)
        return lax.cond(is_edge,
                        lambda b: slow_tile(datar, segr, b),
                        lambda b: fast_tile(datar, segr, b), base)

    @pl.when(nt > 0)
    def _():
        start_tile(0, d0, s0, sem0)

    def tile_body(t, base):
        def even(b):
            wait_tile(t, d0, s0, sem0)
            @pl.when(t + 1 < nt)
            def _():
                start_tile(t + 1, d1, s1, sem1)
            return process_tile(t, d0, s0, b)

        def odd(b):
            wait_tile(t, d1, s1, sem1)
            @pl.when(t + 1 < nt)
            def _():
                start_tile(t + 1, d0, s0, sem0)
            return process_tile(t, d1, s1, b)

        return lax.cond(t % 2 == 0, even, odd, base)

    base = lax.fori_loop(0, nt, tile_body, vlo)

    copy_window(base)
    zero_acc()
    zero_windows(base + KA, (vhi - base - KA) // KA)


def _pool_max_body(xv_hbm, vseg_hbm, pool_hbm, seen_hbm,
                   acc, seenb, d0, d1, s0, s1, probe, sem0, sem1):
    w = lax.axis_index("s") * NC + lax.axis_index("c")
    plo = w * PB
    phi = plo + PB
    lo = _lower_bound(vseg_hbm, N_VIEWS, probe, plo)
    hi = _lower_bound(vseg_hbm, N_VIEWS, probe, phi)
    lo_t = (lo // TB) * TB
    hi_t = ((hi + TB - 1) // TB) * TB
    nt = (hi_t - lo_t) // TB

    ninf = jnp.full((16,), -jnp.inf, jnp.float32)
    zvec = jnp.zeros((16,), jnp.float32)
    ones = jnp.ones((16,), jnp.float32)

    def init_acc(j, c):
        for u in range(8):
            acc[pl.ds(j * D + u * 16, 16)] = ninf
        seenb[pl.ds(j * 16, 16)] = zvec
        return c

    lax.fori_loop(0, PB, init_acc, 0)

    def upd_pixel(datar, ib, sl):
        off = sl * D
        vs = [datar[pl.ds(ib + g * 16, 16)] for g in range(8)]
        curs = [acc[pl.ds(off + g * 16, 16)] for g in range(8)]
        for g in range(8):
            acc[pl.ds(off + g * 16, 16)] = jnp.maximum(curs[g], vs[g])
        seenb[pl.ds(sl * 16, 16)] = ones

    def slow_tile(datar, segr):
        def body(i, c):
            s = _sread(segr, i)
            valid = (s >= plo) & (s < phi)
            sl = jnp.where(valid, s - plo, PB)
            upd_pixel(datar, i * D, sl)
            return c
        lax.fori_loop(0, TB, body, 0)

    def fast_tile(datar, segr):
        def block(bi, c):
            vec = segr[pl.ds(bi * 16, 16)]
            for j in range(16):
                upd_pixel(datar, (bi * 16 + j) * D, vec[j] - plo)
            return c
        lax.fori_loop(0, NBB, block, 0)

    def start_tile(t, datar, segr, sem):
        p0 = pl.multiple_of(lo_t + t * TB, 16)
        pltpu.async_copy(xv_hbm.at[pl.ds(p0 * D, TB * D)], datar, sem)
        pltpu.async_copy(vseg_hbm.at[pl.ds(p0, TB)], segr.at[pl.ds(0, TB)], sem)

    def wait_tile(t, datar, segr, sem):
        p0 = pl.multiple_of(lo_t + t * TB, 16)
        pltpu.make_async_copy(xv_hbm.at[pl.ds(p0 * D, TB * D)], datar, sem).wait()
        pltpu.make_async_copy(vseg_hbm.at[pl.ds(p0, TB)], segr.at[pl.ds(0, TB)], sem).wait()

    def process_tile(t, datar, segr):
        is_edge = (t == 0) | (t == nt - 1)
        lax.cond(is_edge,
                 lambda _: slow_tile(datar, segr),
                 lambda _: fast_tile(datar, segr), 0)

    @pl.when(nt > 0)
    def _():
        start_tile(0, d0, s0, sem0)

    def tile_body(t, c):
        def even(cc):
            wait_tile(t, d0, s0, sem0)
            @pl.when(t + 1 < nt)
            def _():
                start_tile(t + 1, d1, s1, sem1)
            process_tile(t, d0, s0)
            return cc

        def odd(cc):
            wait_tile(t, d1, s1, sem1)
            @pl.when(t + 1 < nt)
            def _():
                start_tile(t + 1, d0, s0, sem0)
            process_tile(t, d1, s1)
            return cc

        return lax.cond(t % 2 == 0, even, odd, c)

    lax.fori_loop(0, nt, tile_body, 0)

    def blend(j, c):
        sv = seenb[pl.ds(j * 16, 16)]
        seen = sv > 0.0
        for u in range(8):
            val = acc[pl.ds(j * D + u * 16, 16)]
            acc[pl.ds(j * D + u * 16, 16)] = jnp.where(seen, val, 0.0)
        return c

    lax.fori_loop(0, PB, blend, 0)

    offp = pl.multiple_of(plo * D, 16)
    pltpu.sync_copy(acc.at[pl.ds(0, PB * D)], pool_hbm.at[pl.ds(offp, PB * D)])
    offs = pl.multiple_of(plo * 16, 16)
    pltpu.sync_copy(seenb.at[pl.ds(0, PB * 16)], seen_hbm.at[pl.ds(offs, PB * 16)])


def _build():
    mesh = plsc.VectorSubcoreMesh(core_axis_name="c", subcore_axis_name="s",
                                  num_cores=NC, num_subcores=NS)
    pool_sum = pl.kernel(
        _pool_sum_body,
        out_type=(
            jax.ShapeDtypeStruct((N_VIEWS * D,), jnp.float32),
            jax.ShapeDtypeStruct((N_VIEWS * 16,), jnp.float32),
        ),
        mesh=mesh,
        scratch_types=[
            pltpu.VMEM(((KA + 1) * D,), jnp.float32),
            pltpu.VMEM(((KA + 1) * 16,), jnp.float32),
            pltpu.VMEM((TA * D,), jnp.float32),
            pltpu.VMEM((TA * D,), jnp.float32),
            pltpu.VMEM((TA + 16,), jnp.int32),
            pltpu.VMEM((TA + 16,), jnp.int32),
            pltpu.VMEM((32,), jnp.int32),
            pltpu.SemaphoreType.DMA,
            pltpu.SemaphoreType.DMA,
        ],
    )
    pool_max = pl.kernel(
        _pool_max_body,
        out_type=(
            jax.ShapeDtypeStruct((NPOINT_PAD * D,), jnp.float32),
            jax.ShapeDtypeStruct((NPOINT_PAD * 16,), jnp.float32),
        ),
        mesh=mesh,
        scratch_types=[
            pltpu.VMEM(((PB + 1) * D,), jnp.float32),
            pltpu.VMEM(((PB + 1) * 16,), jnp.float32),
            pltpu.VMEM((TB * D,), jnp.float32),
            pltpu.VMEM((TB * D,), jnp.float32),
            pltpu.VMEM((TB + 16,), jnp.int32),
            pltpu.VMEM((TB + 16,), jnp.int32),
            pltpu.VMEM((32,), jnp.int32),
            pltpu.SemaphoreType.DMA,
            pltpu.SemaphoreType.DMA,
        ],
    )
    return pool_sum, pool_max


BM = 3200


def _mm_body(sums_ref, cnt_ref, w_ref, b_ref, out_ref):
    c = cnt_ref[:, :1]
    mean = sums_ref[:].reshape(BM, D) / jnp.maximum(c, 1.0)
    y = jnp.dot(mean, w_ref[:], preferred_element_type=jnp.float32)
    y = y + b_ref[:] * (c > 0.0).astype(jnp.float32)
    out_ref[:] = y.reshape(BM * D)


_mm = pl.pallas_call(
    _mm_body,
    grid=(N_VIEWS // BM,),
    in_specs=[
        pl.BlockSpec((BM * D,), lambda i: (i,)),
        pl.BlockSpec((BM, 16), lambda i: (i, 0)),
        pl.BlockSpec((D, D), lambda i: (0, 0)),
        pl.BlockSpec((1, D), lambda i: (0, 0)),
    ],
    out_specs=pl.BlockSpec((BM * D,), lambda i: (i,)),
    out_shape=jax.ShapeDtypeStruct((N_VIEWS * D,), jnp.float32),
)


def kernel(x_3d, x_mod, atomic_seg, view_seg, W, b):
    pool_sum, pool_max = _build()
    seg_a = atomic_seg.astype(jnp.int32)
    seg_v = view_seg.astype(jnp.int32)
    sums_flat, cnt_flat = pool_sum(x_mod.reshape(-1), seg_a)
    xv_flat = _mm(sums_flat, cnt_flat.reshape(N_VIEWS, 16), W, b.reshape(1, D))
    pool_flat, seen_flat = pool_max(xv_flat, seg_v)
    x_pool = pool_flat.reshape(NPOINT_PAD, D)[:N_POINTS]
    x_seen = seen_flat.reshape(NPOINT_PAD, 16)[:N_POINTS, 0] > 0.0
    out = jnp.concatenate([x_3d, x_pool], axis=1)
    return out, x_seen
